# baseline (device time: 24944 ns/iter reference)
import jax
import jax.numpy as jnp
from jax import lax
from jax.experimental import pallas as pl
from jax.experimental.pallas import tpu as pltpu

N_DEV = 8
B, SQ, SKV = 2, 256, 256
HQ_LOCAL, DH = 4, 64
HD = HQ_LOCAL * DH
DM = 512
BLK = 64
ROWS = B * SQ
CH = ROWS // N_DEV

BF = jnp.bfloat16
F32 = jnp.float32


def _attn_group(q, k, v):
    s = lax.dot_general(q, k, (((1,), (1,)), ((), ())),
                        preferred_element_type=F32) * 0.125
    e = jnp.exp(s)
    r = 1.0 / jnp.sum(e, axis=1, keepdims=True)
    c = jnp.dot(e.astype(BF), v, preferred_element_type=F32)
    return c * r


def kernel(x, Wq, K_ext, V_ext, Wo):
    K2 = K_ext.reshape(B, SKV, -1)
    V2 = V_ext.reshape(B, SKV, -1)
    x2 = x.reshape(ROWS, DM)

    def body(x_ref, wq_ref, k_ref, v_ref, wo_ref, out_ref,
             send_buf, recv_buf, bcast_buf, s1, r1, s2, r2):
        my = lax.axis_index("i")
        peers = [lax.rem(my + off, N_DEV) for off in range(1, N_DEV)]

        barrier_sem = pltpu.get_barrier_semaphore()
        for p in peers:
            pl.semaphore_signal(barrier_sem, inc=1, device_id=(p,),
                                device_id_type=pl.DeviceIdType.MESH)
        pl.semaphore_wait(barrier_sem, N_DEV - 1)

        col0 = my * HD

        wq16 = wq_ref[:, :].astype(BF)
        q_full = jnp.dot(x_ref[:, :].astype(BF), wq16,
                         preferred_element_type=F32).astype(BF)

        ctx_b = []
        for b in range(B):
            k_all = k_ref[b, :, pl.ds(col0, HD)].astype(BF)
            v_all = v_ref[b, :, pl.ds(col0, HD)].astype(BF)
            heads = []
            for h in range(HQ_LOCAL):
                qh = q_full[b * SQ:(b + 1) * SQ, h * DH:(h + 1) * DH]
                kh = k_all[:, h * DH:(h + 1) * DH]
                vh = v_all[:, h * DH:(h + 1) * DH]
                qa = jnp.concatenate([qh[0:BLK], qh[3 * BLK:4 * BLK]], axis=0)
                ka = jnp.concatenate([kh[0:BLK], kh[3 * BLK:4 * BLK]], axis=0)
                va = jnp.concatenate([vh[0:BLK], vh[3 * BLK:4 * BLK]], axis=0)
                ca = _attn_group(qa, ka, va)
                cb = _attn_group(qh[BLK:3 * BLK],
                                 kh[0:3 * BLK], vh[0:3 * BLK])
                ctx = jnp.concatenate(
                    [ca[0:BLK], cb, ca[BLK:2 * BLK]], axis=0)
                heads.append(ctx.astype(BF))
            ctx_b.append(jnp.concatenate(heads, axis=1))
        ctxc = jnp.concatenate(ctx_b, axis=0)
        partial = jnp.dot(ctxc, wo_ref[:, :].astype(BF),
                          preferred_element_type=F32)
        for j in range(N_DEV):
            send_buf[j, :, :] = partial[j * CH:(j + 1) * CH, :].astype(BF)

        p1 = []
        for p in peers:
            d = pltpu.make_async_remote_copy(
                src_ref=send_buf.at[p],
                dst_ref=recv_buf.at[my],
                send_sem=s1.at[p],
                recv_sem=r1.at[my],
                device_id=(p,),
                device_id_type=pl.DeviceIdType.MESH,
            )
            d.start()
            p1.append(d)

        acc = send_buf[my].astype(F32)
        for k in range(N_DEV - 1):
            p = lax.rem(my - 1 - k + 2 * N_DEV, N_DEV)
            pltpu.make_async_remote_copy(
                src_ref=send_buf.at[p], dst_ref=recv_buf.at[p],
                send_sem=s1.at[p], recv_sem=r1.at[p],
                device_id=(p,), device_id_type=pl.DeviceIdType.MESH,
            ).wait_recv()
            acc = acc + recv_buf[p].astype(F32)
        acc16 = acc.astype(BF)
        bcast_buf[:, :] = acc16
        out_ref[pl.ds(my * CH, CH), :] = acc16

        for d in p1:
            d.wait_send()

        p2 = []
        for off in range(1, N_DEV):
            p = lax.rem(my + off, N_DEV)
            d = pltpu.make_async_remote_copy(
                src_ref=bcast_buf,
                dst_ref=out_ref.at[pl.ds(my * CH, CH), :],
                send_sem=s2.at[off],
                recv_sem=r2.at[my],
                device_id=(p,),
                device_id_type=pl.DeviceIdType.MESH,
            )
            d.start()
            p2.append(d)

        for k in range(N_DEV - 1):
            p = lax.rem(my - 1 - k + 2 * N_DEV, N_DEV)
            pltpu.make_async_remote_copy(
                src_ref=bcast_buf, dst_ref=out_ref.at[pl.ds(p * CH, CH), :],
                send_sem=s2.at[k], recv_sem=r2.at[p],
                device_id=(p,), device_id_type=pl.DeviceIdType.MESH,
            ).wait_recv()

        for d in p2:
            d.wait_send()

    out2d = pl.pallas_call(
        body,
        out_shape=jax.ShapeDtypeStruct((ROWS, DM), BF),
        in_specs=[pl.BlockSpec(memory_space=pltpu.VMEM)] * 5,
        out_specs=pl.BlockSpec(memory_space=pltpu.VMEM),
        scratch_shapes=[
            pltpu.VMEM((N_DEV, CH, DM), BF),
            pltpu.VMEM((N_DEV, CH, DM), BF),
            pltpu.VMEM((CH, DM), BF),
            pltpu.SemaphoreType.DMA((N_DEV,)),
            pltpu.SemaphoreType.DMA((N_DEV,)),
            pltpu.SemaphoreType.DMA((N_DEV,)),
            pltpu.SemaphoreType.DMA((N_DEV,)),
        ],
        compiler_params=pltpu.CompilerParams(collective_id=0),
    )(x2, Wq, K2, V2, Wo)
    return out2d.reshape(B, SQ, DM)
